# Initial kernel scaffold; baseline (speedup 1.0000x reference)
import functools

import jax
import jax.numpy as jnp
from jax.experimental import pallas as pl

_KMAX = 0.2
_KMIN = 0.2
_ALPHA = 0.7
_INT_MIN = -2147483648


def _pos_k(k, n):
    if k <= 0:
        return 0
    elif k < 1:
        return int(round(k * n))
    elif k > n:
        return int(n)
    return int(k)


def _key_fwd(i):
    return jnp.where(i >= 0, i, i ^ jnp.int32(0x7FFFFFFF))


def _body(k, alpha, x_ref, o_ref):
    x = x_ref[...]
    i = jax.lax.bitcast_convert_type(x, jnp.int32)
    ikey = _key_fwd(i)
    jkey = ~ikey
    kk = jnp.int32(k)
    r = x.shape[0]
    init = jnp.full((r, 1), jnp.int32(_INT_MIN))

    def bit_body(t, carry):
        p_hi, p_lo = carry
        bit = jax.lax.shift_left(jnp.int32(1), jnp.int32(30) - t)
        c_hi = p_hi + bit
        c_lo = p_lo + bit
        cnt_hi = jnp.sum((ikey >= c_hi).astype(jnp.int32), axis=1,
                         keepdims=True)
        cnt_lo = jnp.sum((jkey >= c_lo).astype(jnp.int32), axis=1,
                         keepdims=True)
        p_hi = jnp.where(cnt_hi >= kk, c_hi, p_hi)
        p_lo = jnp.where(cnt_lo >= kk, c_lo, p_lo)
        return p_hi, p_lo

    p_hi, p_lo = jax.lax.fori_loop(0, 31, bit_body, (init, init))

    def key_to_f32(kv):
        return jax.lax.bitcast_convert_type(_key_fwd(kv), jnp.float32)

    t_hi = key_to_f32(p_hi)
    t_lo = key_to_f32(~p_lo)
    kf = jnp.float32(k)
    s_top = kf * t_hi[:, 0] + jnp.sum(jnp.maximum(x - t_hi, 0.0), axis=1)
    s_bot = kf * t_lo[:, 0] - jnp.sum(jnp.maximum(t_lo - x, 0.0), axis=1)
    out = (s_top + jnp.float32(alpha) * s_bot) * jnp.float32(1.0 / (2.0 * k))
    o_ref[...] = out.reshape(r, 1)


def kernel(input):
    b, c, h, w = input.shape
    n = h * w
    kmax = _pos_k(_KMAX, n)
    num_rows = b * c
    flat = input.reshape(num_rows, n)
    r = 512
    assert num_rows % r == 0
    out = pl.pallas_call(
        functools.partial(_body, kmax, _ALPHA),
        grid=(num_rows // r,),
        in_specs=[pl.BlockSpec((r, n), lambda i: (i, 0))],
        out_specs=pl.BlockSpec((r, 1), lambda i: (i, 0)),
        out_shape=jax.ShapeDtypeStruct((num_rows, 1), jnp.float32),
    )(flat)
    return out.reshape(b, c)


# speedup vs baseline: 3.4465x; 3.4465x over previous
import functools

import jax
import jax.numpy as jnp
from jax.experimental import pallas as pl

_KMAX = 0.2
_KMIN = 0.2
_ALPHA = 0.7
_INT_MIN = -2147483648


def _pos_k(k, n):
    if k <= 0:
        return 0
    elif k < 1:
        return int(round(k * n))
    elif k > n:
        return int(n)
    return int(k)


def _key_fwd(i):
    return jnp.where(i >= 0, i, i ^ jnp.int32(0x7FFFFFFF))


def _body(k, alpha, x_ref, o_ref):
    x = x_ref[...]
    i = jax.lax.bitcast_convert_type(x, jnp.int32)
    ikey = _key_fwd(i)
    jkey = ~ikey
    kk = jnp.int32(k)
    r = x.shape[0]
    imin = jnp.int32(_INT_MIN)

    def sign_init(key):
        cnt0 = jnp.sum((key >= 0).astype(jnp.int32), axis=1, keepdims=True)
        return jnp.where(cnt0 >= kk, jnp.int32(0), imin)

    init_hi = sign_init(ikey)
    init_lo = sign_init(jkey)

    def bit_body(t, carry):
        p_hi, p_lo = carry
        bit = jax.lax.shift_left(jnp.int32(1), jnp.int32(30) - t)
        c_hi = p_hi + bit
        c_lo = p_lo + bit
        cnt_hi = jnp.sum((ikey >= c_hi).astype(jnp.int32), axis=1,
                         keepdims=True)
        cnt_lo = jnp.sum((jkey >= c_lo).astype(jnp.int32), axis=1,
                         keepdims=True)
        p_hi = jnp.where(cnt_hi >= kk, c_hi, p_hi)
        p_lo = jnp.where(cnt_lo >= kk, c_lo, p_lo)
        return p_hi, p_lo

    p_hi, p_lo = jax.lax.fori_loop(0, 31, bit_body, (init_hi, init_lo))

    def key_to_f32(kv):
        return jax.lax.bitcast_convert_type(_key_fwd(kv), jnp.float32)

    t_hi = key_to_f32(p_hi)
    t_lo = key_to_f32(~p_lo)
    kf = jnp.float32(k)
    s_top = kf * t_hi[:, 0] + jnp.sum(jnp.maximum(x - t_hi, 0.0), axis=1)
    s_bot = kf * t_lo[:, 0] - jnp.sum(jnp.maximum(t_lo - x, 0.0), axis=1)
    out = (s_top + jnp.float32(alpha) * s_bot) * jnp.float32(1.0 / (2.0 * k))
    o_ref[...] = out.reshape(r, 1)


def kernel(input):
    b, c, h, w = input.shape
    n = h * w
    kmax = _pos_k(_KMAX, n)
    num_rows = b * c
    flat = input.reshape(num_rows, n)
    r = 512
    assert num_rows % r == 0
    out = pl.pallas_call(
        functools.partial(_body, kmax, _ALPHA),
        grid=(num_rows // r,),
        in_specs=[pl.BlockSpec((r, n), lambda i: (i, 0))],
        out_specs=pl.BlockSpec((r, 1), lambda i: (i, 0)),
        out_shape=jax.ShapeDtypeStruct((num_rows, 1), jnp.float32),
    )(flat)
    return out.reshape(b, c)


# speedup vs baseline: 3.4608x; 1.0042x over previous
import functools

import jax
import jax.numpy as jnp
from jax.experimental import pallas as pl

_KMAX = 0.2
_KMIN = 0.2
_ALPHA = 0.7
_INT_MIN = -2147483648


def _pos_k(k, n):
    if k <= 0:
        return 0
    elif k < 1:
        return int(round(k * n))
    elif k > n:
        return int(n)
    return int(k)


def _key_fwd(i):
    return jnp.where(i >= 0, i, i ^ jnp.int32(0x7FFFFFFF))


def _body(k, alpha, x_ref, o_ref):
    x = x_ref[...]
    i = jax.lax.bitcast_convert_type(x, jnp.int32)
    ikey = _key_fwd(i)
    kk = jnp.int32(k)
    r = x.shape[0]
    imin = jnp.int32(_INT_MIN)

    def counts(c_hi, c_lo):
        v = ((ikey >= c_hi).astype(jnp.int32)
             + jax.lax.shift_left((ikey <= ~c_lo).astype(jnp.int32),
                                  jnp.int32(16)))
        s = jnp.sum(v, axis=1, keepdims=True)
        return s & jnp.int32(0xFFFF), jax.lax.shift_right_logical(
            s, jnp.int32(16))

    cnt0_hi, cnt0_lo = counts(jnp.int32(0), jnp.int32(0))
    init_hi = jnp.where(cnt0_hi >= kk, jnp.int32(0), imin)
    init_lo = jnp.where(cnt0_lo >= kk, jnp.int32(0), imin)

    def bit_body(t, carry):
        p_hi, p_lo = carry
        bit = jax.lax.shift_left(jnp.int32(1), jnp.int32(30) - t)
        c_hi = p_hi + bit
        c_lo = p_lo + bit
        cnt_hi, cnt_lo = counts(c_hi, c_lo)
        p_hi = jnp.where(cnt_hi >= kk, c_hi, p_hi)
        p_lo = jnp.where(cnt_lo >= kk, c_lo, p_lo)
        return p_hi, p_lo

    p_hi, p_lo = jax.lax.fori_loop(0, 31, bit_body, (init_hi, init_lo))

    def key_to_f32(kv):
        return jax.lax.bitcast_convert_type(_key_fwd(kv), jnp.float32)

    t_hi = key_to_f32(p_hi)
    t_lo = key_to_f32(~p_lo)
    kf = jnp.float32(k)
    s_top = kf * t_hi[:, 0] + jnp.sum(jnp.maximum(x - t_hi, 0.0), axis=1)
    s_bot = kf * t_lo[:, 0] - jnp.sum(jnp.maximum(t_lo - x, 0.0), axis=1)
    out = (s_top + jnp.float32(alpha) * s_bot) * jnp.float32(1.0 / (2.0 * k))
    o_ref[...] = out.reshape(r, 1)


def kernel(input):
    b, c, h, w = input.shape
    n = h * w
    kmax = _pos_k(_KMAX, n)
    num_rows = b * c
    flat = input.reshape(num_rows, n)
    r = 512
    assert num_rows % r == 0
    out = pl.pallas_call(
        functools.partial(_body, kmax, _ALPHA),
        grid=(num_rows // r,),
        in_specs=[pl.BlockSpec((r, n), lambda i: (i, 0))],
        out_specs=pl.BlockSpec((r, 1), lambda i: (i, 0)),
        out_shape=jax.ShapeDtypeStruct((num_rows, 1), jnp.float32),
    )(flat)
    return out.reshape(b, c)
